# baseline (speedup 1.0000x reference)
import functools

import jax
import jax.numpy as jnp
from jax import lax
from jax.experimental import pallas as pl
from jax.experimental.pallas import tpu as pltpu
from jax.experimental.pallas import tpu_sc as plsc

_B = 16384
_D = 64
_NW = 32
_EPW = _B // _NW
_C = 128
_NCH = _EPW // _C
_L = 16


def _complex_body(head_hbm, rel_hbm, tail_hbm,
                  er_hbm, ei_hbm, rr_hbm, ri_hbm, out_hbm,
                  h_iv, r_iv, t_iv, out_v,
                  hr_b, hi_b, tr_b, ti_b, rr_b, ri_b, sem):
    wid = lax.axis_index("s") * 2 + lax.axis_index("c")
    base = wid * _EPW

    iota = lax.iota(jnp.int32, _L)

    for ch in range(_NCH):
        pltpu.sync_copy(head_hbm.at[pl.ds(base + ch * _C, _C)], h_iv)
        pltpu.sync_copy(rel_hbm.at[pl.ds(base + ch * _C, _C)], r_iv)
        pltpu.sync_copy(tail_hbm.at[pl.ds(base + ch * _C, _C)], t_iv)

        def issue(g, _):
            hv = h_iv[pl.ds(g * _L, _L)]
            rv = r_iv[pl.ds(g * _L, _L)]
            tv = t_iv[pl.ds(g * _L, _L)]
            for lane in range(_L):
                m = iota == lane
                h = jnp.sum(jnp.where(m, hv, 0))
                r = jnp.sum(jnp.where(m, rv, 0))
                t = jnp.sum(jnp.where(m, tv, 0))
                e = g * _L + lane
                pltpu.async_copy(er_hbm.at[h], hr_b.at[e], sem)
                pltpu.async_copy(ei_hbm.at[h], hi_b.at[e], sem)
                pltpu.async_copy(er_hbm.at[t], tr_b.at[e], sem)
                pltpu.async_copy(ei_hbm.at[t], ti_b.at[e], sem)
                pltpu.async_copy(rr_hbm.at[r], rr_b.at[e], sem)
                pltpu.async_copy(ri_hbm.at[r], ri_b.at[e], sem)
            return 0

        lax.fori_loop(0, _C // _L, issue, 0)

        for buf in (hr_b, hi_b, tr_b, ti_b, rr_b, ri_b):
            pltpu.make_async_copy(er_hbm.at[pl.ds(0, _C)], buf, sem).wait()

        def grp_body(g, _, ch=ch):
            rows = g * _L + iota

            def dim_body(d, acc):
                cols = jnp.zeros((_L,), jnp.int32) + d
                hr = plsc.load_gather(hr_b, [rows, cols])
                hi = plsc.load_gather(hi_b, [rows, cols])
                tr = plsc.load_gather(tr_b, [rows, cols])
                ti = plsc.load_gather(ti_b, [rows, cols])
                rr = plsc.load_gather(rr_b, [rows, cols])
                ri = plsc.load_gather(ri_b, [rows, cols])
                return acc + tr * (hr * rr - hi * ri) + ti * (hi * rr + hr * ri)

            acc = lax.fori_loop(0, _D, dim_body, jnp.zeros((_L,), jnp.float32))
            out_v[pl.ds(ch * _C + g * _L, _L)] = acc
            return 0

        lax.fori_loop(0, _C // _L, grp_body, 0)

    pltpu.sync_copy(out_v, out_hbm.at[pl.ds(base, _EPW)])


@jax.jit
def kernel(head, relation, tail, ent_real, ent_imag, rel_real, rel_imag):
    mesh = plsc.VectorSubcoreMesh(core_axis_name="c", subcore_axis_name="s")
    run = pl.kernel(
        _complex_body,
        out_type=jax.ShapeDtypeStruct((_B,), jnp.float32),
        mesh=mesh,
        scratch_types=[
            pltpu.VMEM((_C,), jnp.int32),
            pltpu.VMEM((_C,), jnp.int32),
            pltpu.VMEM((_C,), jnp.int32),
            pltpu.VMEM((_EPW,), jnp.float32),
            pltpu.VMEM((_C, _D), jnp.float32),
            pltpu.VMEM((_C, _D), jnp.float32),
            pltpu.VMEM((_C, _D), jnp.float32),
            pltpu.VMEM((_C, _D), jnp.float32),
            pltpu.VMEM((_C, _D), jnp.float32),
            pltpu.VMEM((_C, _D), jnp.float32),
            pltpu.SemaphoreType.DMA,
        ],
        compiler_params=pltpu.CompilerParams(needs_layout_passes=False),
    )
    return run(head, relation, tail, ent_real, ent_imag, rel_real, rel_imag)


# speedup vs baseline: 1.0003x; 1.0003x over previous
import functools

import jax
import jax.numpy as jnp
from jax import lax
from jax.experimental import pallas as pl
from jax.experimental.pallas import tpu as pltpu
from jax.experimental.pallas import tpu_sc as plsc

_B = 16384
_D = 64
_NW = 32
_EPW = _B // _NW
_C = 128
_NCH = _EPW // _C
_L = 16


def _complex_body(head_hbm, rel_hbm, tail_hbm,
                  er_hbm, ei_hbm, rr_hbm, ri_hbm, out_hbm,
                  h_iv, r_iv, t_iv, out_v,
                  hr_b, hi_b, tr_b, ti_b, rr_b, ri_b, sem):
    wid = lax.axis_index("s") * 2 + lax.axis_index("c")
    base = wid * _EPW

    iota = lax.iota(jnp.int32, _L)

    for ch in range(_NCH):
        pltpu.sync_copy(head_hbm.at[pl.ds(base + ch * _C, _C)], h_iv)
        pltpu.sync_copy(rel_hbm.at[pl.ds(base + ch * _C, _C)], r_iv)
        pltpu.sync_copy(tail_hbm.at[pl.ds(base + ch * _C, _C)], t_iv)

        def issue(g, _):
            hv = h_iv[pl.ds(g * _L, _L)]
            rv = r_iv[pl.ds(g * _L, _L)]
            tv = t_iv[pl.ds(g * _L, _L)]
            for lane in range(_L):
                m = iota == lane
                h = jnp.sum(jnp.where(m, hv, 0))
                r = jnp.sum(jnp.where(m, rv, 0))
                t = jnp.sum(jnp.where(m, tv, 0))
                e = g * _L + lane
                pltpu.async_copy(er_hbm.at[h], hr_b.at[e], sem)
                pltpu.async_copy(ei_hbm.at[h], hi_b.at[e], sem)
                pltpu.async_copy(er_hbm.at[t], tr_b.at[e], sem)
                pltpu.async_copy(ei_hbm.at[t], ti_b.at[e], sem)
                pltpu.async_copy(rr_hbm.at[r], rr_b.at[e], sem)
                pltpu.async_copy(ri_hbm.at[r], ri_b.at[e], sem)
            return 0

        lax.fori_loop(0, _C // _L, issue, 0)

        for buf in (hr_b, hi_b, tr_b, ti_b, rr_b, ri_b):
            pltpu.make_async_copy(er_hbm.at[pl.ds(0, _C)], buf, sem).wait()

        def grp_body(g, _, ch=ch):
            rows = g * _L + iota

            def dim_body(d, acc):
                cols = jnp.zeros((_L,), jnp.int32) + d
                hr = plsc.load_gather(hr_b, [rows, cols])
                hi = plsc.load_gather(hi_b, [rows, cols])
                tr = plsc.load_gather(tr_b, [rows, cols])
                ti = plsc.load_gather(ti_b, [rows, cols])
                rr = plsc.load_gather(rr_b, [rows, cols])
                ri = plsc.load_gather(ri_b, [rows, cols])
                return acc + tr * (hr * rr - hi * ri) + ti * (hi * rr + hr * ri)

            acc = lax.fori_loop(0, _D, dim_body, jnp.zeros((_L,), jnp.float32))
            out_v[pl.ds(ch * _C + g * _L, _L)] = acc
            return 0

        lax.fori_loop(0, _C // _L, grp_body, 0)

    pltpu.sync_copy(out_v, out_hbm.at[pl.ds(base, _EPW)])


@jax.jit
def kernel(head, relation, tail, ent_real, ent_imag, rel_real, rel_imag):
    mesh = plsc.VectorSubcoreMesh(core_axis_name="c", subcore_axis_name="s")
    run = pl.kernel(
        _complex_body,
        out_type=jax.ShapeDtypeStruct((_B,), jnp.float32),
        mesh=mesh,
        scratch_types=[
            pltpu.VMEM((_C,), jnp.int32),
            pltpu.VMEM((_C,), jnp.int32),
            pltpu.VMEM((_C,), jnp.int32),
            pltpu.VMEM((_EPW,), jnp.float32),
            pltpu.VMEM((_C, _D), jnp.float32),
            pltpu.VMEM((_C, _D), jnp.float32),
            pltpu.VMEM((_C, _D), jnp.float32),
            pltpu.VMEM((_C, _D), jnp.float32),
            pltpu.VMEM((_C, _D), jnp.float32),
            pltpu.VMEM((_C, _D), jnp.float32),
            pltpu.SemaphoreType.DMA,
        ],
        compiler_params=pltpu.CompilerParams(
            needs_layout_passes=False, use_tc_tiling_on_sc=True),
    )
    return run(head, relation, tail, ent_real, ent_imag, rel_real, rel_imag)
